# Initial kernel scaffold; baseline (speedup 1.0000x reference)
import jax
import jax.numpy as jnp
from jax.experimental import pallas as pl


def kernel(inputs, weight):
    raise NotImplementedError("write your pallas kernel here")


# speedup vs baseline: 1.5762x; 1.5762x over previous
import functools

import jax
import jax.numpy as jnp
from jax import lax
from jax.experimental import pallas as pl
from jax.experimental.pallas import tpu as pltpu
from jax.experimental.pallas import tpu_sc as plsc

NUM_EMB = 1000000
DIM = 32

NC = 2
NS = 16
NW = NC * NS

B_TOTAL = 16384 * 26
R = B_TOTAL // NW
CH = 1024
NCHUNK = R // CH
assert NCHUNK * CH == R


def _body(idx_hbm, table_hbm, out_hbm, idx_v, buf0, buf1,
          gsem0, gsem1, ssem0, ssem1):
    wid = lax.axis_index("s") * NC + lax.axis_index("c")
    base = wid * R
    pltpu.sync_copy(idx_hbm.at[wid], idx_v)

    bufs = (buf0, buf1)
    gsems = (gsem0, gsem1)
    ssems = (ssem0, ssem1)
    g = [None, None]
    s = [None, None]

    g[0] = pltpu.async_copy(
        table_hbm.at[idx_v.at[pl.ds(0, CH)]], bufs[0], gsems[0])
    for c in range(NCHUNK):
        b = c & 1
        nb = b ^ 1
        if c + 1 < NCHUNK:
            if s[nb] is not None:
                s[nb].wait()
            g[nb] = pltpu.async_copy(
                table_hbm.at[idx_v.at[pl.ds((c + 1) * CH, CH)]],
                bufs[nb], gsems[nb])
        g[b].wait()
        s[b] = pltpu.async_copy(
            bufs[b], out_hbm.at[pl.ds(base + c * CH, CH)], ssems[b])
    for b in range(2):
        if s[b] is not None:
            s[b].wait()


@functools.partial(jax.jit, static_argnames=())
def _gather_flat(idx, table):
    mesh = plsc.VectorSubcoreMesh(core_axis_name="c", subcore_axis_name="s")
    k = pl.kernel(
        _body,
        mesh=mesh,
        compiler_params=pltpu.CompilerParams(use_tc_tiling_on_sc=False),
        out_type=jax.ShapeDtypeStruct((B_TOTAL, DIM), jnp.float32),
        scratch_types=[
            pltpu.VMEM((R,), jnp.int32),
            pltpu.VMEM((CH, DIM), jnp.float32),
            pltpu.VMEM((CH, DIM), jnp.float32),
            pltpu.SemaphoreType.DMA,
            pltpu.SemaphoreType.DMA,
            pltpu.SemaphoreType.DMA,
            pltpu.SemaphoreType.DMA,
        ],
    )
    return k(idx, table)


def kernel(inputs, weight):
    idx = inputs.astype(jnp.int32).reshape(NW, R)
    out = _gather_flat(idx, weight)
    return out.reshape(inputs.shape + (DIM,))
